# baseline (speedup 1.0000x reference)
import functools

import jax
import jax.numpy as jnp
from jax import lax
from jax.experimental import pallas as pl
from jax.experimental.pallas import tpu as pltpu
from jax.experimental.pallas import tpu_sc as plsc

_H = 128
_ROWS_PER_STREAM = 128
_STREAMS_PER_SUPER = 5
_SL = 16


def _sc_gather(table, idx2d):
    n_chunks = idx2d.shape[0]
    d = table.shape[1]
    info = plsc.get_sparse_core_info()
    nc, ns = info.num_cores, info.num_subcores
    nw = nc * ns
    ch_per_w = n_chunks // nw
    n_super = ch_per_w // _STREAMS_PER_SUPER
    assert n_chunks % nw == 0 and ch_per_w % _STREAMS_PER_SUPER == 0

    mesh = plsc.VectorSubcoreMesh(core_axis_name="c", subcore_axis_name="s")
    rows_per_super = _STREAMS_PER_SUPER * _ROWS_PER_STREAM
    g_per_stream = _ROWS_PER_STREAM // _SL
    g_per_super = rows_per_super // _SL

    idx4d = idx2d.reshape(nw, n_super, _STREAMS_PER_SUPER, _ROWS_PER_STREAM)
    n_rows = n_chunks * _ROWS_PER_STREAM

    @functools.partial(
        pl.kernel,
        mesh=mesh,
        out_type=jax.ShapeDtypeStruct((n_rows, d), jnp.int32),
        scratch_types=[
            pltpu.VMEM((_STREAMS_PER_SUPER, _ROWS_PER_STREAM), jnp.int32),
            pltpu.VMEM((rows_per_super, d), jnp.int32),
            pltpu.SemaphoreType.DMA,
        ],
    )
    def k(table_hbm, idx_hbm, out_hbm, idx_v, rows_v, sem):
        wid = lax.axis_index("s") * nc + lax.axis_index("c")
        base_ch = wid * ch_per_w

        def body(s, carry):
            ch0 = base_ch + s * _STREAMS_PER_SUPER
            pltpu.sync_copy(idx_hbm.at[wid, s], idx_v)
            copies = [
                pltpu.async_copy(
                    table_hbm.at[idx_v.at[j]],
                    rows_v.at[pl.ds(j * _ROWS_PER_STREAM, _ROWS_PER_STREAM)],
                    sem,
                )
                for j in range(_STREAMS_PER_SUPER)
            ]
            for cp in copies:
                cp.wait()
            pltpu.sync_copy(
                rows_v,
                out_hbm.at[pl.ds(ch0 * _ROWS_PER_STREAM, rows_per_super)],
            )
            return carry

        lax.fori_loop(0, n_super, body, 0)

    return k(table, idx4d)


def _lstm_body(x_ref, p_ref, w2_ref, u_ref, b_ref, wd_ref, bd_ref, o_ref,
               h_ref, c_ref, *, tt, batch, d):
    ti = pl.program_id(0)

    @pl.when(ti == 0)
    def _():
        h_ref[...] = jnp.zeros((batch, _H), jnp.float32)
        c_ref[...] = jnp.zeros((batch, _H), jnp.float32)

    sub = (lax.broadcasted_iota(jnp.int32, (batch, d), 1)
           // (d // 4)).astype(jnp.bfloat16)
    h = h_ref[...]
    c = c_ref[...]
    for k in range(tt):
        m = p_ref[0, :, k:k + 1]
        mask = (sub == m).astype(jnp.bfloat16)
        xw = x_ref[k] * mask
        z = jnp.dot(xw, w2_ref[...], preferred_element_type=jnp.float32)
        z = z + jnp.dot(h.astype(jnp.bfloat16), u_ref[...],
                        preferred_element_type=jnp.float32)
        z = z + b_ref[...]
        i = jax.nn.sigmoid(z[:, :_H])
        f = jax.nn.sigmoid(z[:, _H:2 * _H])
        g = jnp.tanh(z[:, 2 * _H:3 * _H])
        o = jax.nn.sigmoid(z[:, 3 * _H:])
        c = f * c + i * g
        h = o * jnp.tanh(c)
        o_ref[k, 0, :] = jax.nn.sigmoid(
            jnp.sum(h * wd_ref[...], axis=1) + bd_ref[0, 0])
    h_ref[...] = h
    c_ref[...] = c


def _lstm_fused(x2, parT, w2, u, b2, wdp, bdp, tt=8):
    t, batch, d = x2.shape
    grid = (t // tt,)
    par3 = jnp.transpose(parT.reshape(batch, t // tt, tt), (1, 0, 2))
    out = pl.pallas_call(
        functools.partial(_lstm_body, tt=tt, batch=batch, d=d),
        grid=grid,
        in_specs=[
            pl.BlockSpec((tt, batch, d), lambda ti: (ti, 0, 0)),
            pl.BlockSpec((1, batch, tt), lambda ti: (ti, 0, 0)),
            pl.BlockSpec(w2.shape, lambda ti: (0, 0)),
            pl.BlockSpec(u.shape, lambda ti: (0, 0)),
            pl.BlockSpec(b2.shape, lambda ti: (0, 0)),
            pl.BlockSpec(wdp.shape, lambda ti: (0, 0)),
            pl.BlockSpec(memory_space=pltpu.SMEM),
        ],
        out_specs=pl.BlockSpec((tt, 1, batch), lambda ti: (ti, 0, 0)),
        out_shape=jax.ShapeDtypeStruct((t, 1, batch), jnp.float32),
        scratch_shapes=[
            pltpu.VMEM((batch, _H), jnp.float32),
            pltpu.VMEM((batch, _H), jnp.float32),
        ],
        compiler_params=pltpu.CompilerParams(
            dimension_semantics=("arbitrary",)),
    )(x2, par3, w2, u, b2, wdp, bdp)
    return out


def kernel(reviews, emb, W, U, b, gamma, beta, moving_mean, moving_var,
           Wd, bd):
    batch, t = reviews.shape
    v, e = emb.shape
    quads = lax.bitcast_convert_type(
        emb.astype(jnp.bfloat16).reshape(v // 4, 2 * e, 2), jnp.int32)
    idxq = (jnp.transpose(reviews) >> 2).reshape(-1, _ROWS_PER_STREAM)
    parT = (reviews & 3).astype(jnp.bfloat16)
    x4 = _sc_gather(quads, idxq)
    x2 = lax.bitcast_convert_type(
        x4, jnp.bfloat16).reshape(t, batch, 4 * e)

    inv = gamma * lax.rsqrt(moving_var + 1e-3)
    wd0 = Wd[:, 0]
    wdp = (inv * wd0)[None, :]
    bdp = (bd[0] + jnp.sum((beta - inv * moving_mean) * wd0))[None, None]
    b2 = b[None, :]
    w2 = jnp.concatenate([W, W, W, W], axis=0).astype(jnp.bfloat16)

    out = _lstm_fused(x2, parT, w2, U.astype(jnp.bfloat16),
                      b2, wdp, bdp)
    return jnp.transpose(out.reshape(t, batch), (1, 0))[..., None]


# speedup vs baseline: 22.0178x; 22.0178x over previous
import functools

import jax
import jax.numpy as jnp
from jax import lax
from jax.experimental import pallas as pl
from jax.experimental.pallas import tpu as pltpu
from jax.experimental.pallas import tpu_sc as plsc

_H = 128
_ROWS_PER_STREAM = 128
_STREAMS_PER_SUPER = 10


def _sc_gather(table, idx2d):
    n_chunks = idx2d.shape[0]
    e = table.shape[1]
    info = plsc.get_sparse_core_info()
    nc, ns = info.num_cores, info.num_subcores
    nw = nc * ns
    ch_per_w = n_chunks // nw
    n_super = ch_per_w // _STREAMS_PER_SUPER
    assert n_chunks % nw == 0 and ch_per_w % _STREAMS_PER_SUPER == 0

    mesh = plsc.VectorSubcoreMesh(core_axis_name="c", subcore_axis_name="s")
    rows_per_super = _STREAMS_PER_SUPER * _ROWS_PER_STREAM

    idx4d = idx2d.reshape(nw, n_super, _STREAMS_PER_SUPER, _ROWS_PER_STREAM)

    @functools.partial(
        pl.kernel,
        mesh=mesh,
        out_type=jax.ShapeDtypeStruct((n_chunks * _ROWS_PER_STREAM, e),
                                      jnp.float32),
        scratch_types=[
            pltpu.VMEM((_STREAMS_PER_SUPER, _ROWS_PER_STREAM), jnp.int32),
            pltpu.VMEM((rows_per_super, e), jnp.float32),
            pltpu.SemaphoreType.DMA,
        ],
        compiler_params=pltpu.CompilerParams(use_tc_tiling_on_sc=False),
    )
    def k(table_hbm, idx_hbm, out_hbm, idx_v, rows_v, sem):
        wid = lax.axis_index("s") * nc + lax.axis_index("c")
        base_ch = wid * ch_per_w

        def body(s, carry):
            ch0 = base_ch + s * _STREAMS_PER_SUPER
            pltpu.sync_copy(idx_hbm.at[wid, s], idx_v)
            copies = [
                pltpu.async_copy(
                    table_hbm.at[idx_v.at[j]],
                    rows_v.at[pl.ds(j * _ROWS_PER_STREAM, _ROWS_PER_STREAM)],
                    sem,
                )
                for j in range(_STREAMS_PER_SUPER)
            ]
            for cp in copies:
                cp.wait()
            pltpu.sync_copy(
                rows_v,
                out_hbm.at[pl.ds(ch0 * _ROWS_PER_STREAM, rows_per_super)],
            )
            return carry

        lax.fori_loop(0, n_super, body, 0)

    return k(table, idx4d)


def _lstm_body(x_ref, w_ref, u_ref, b_ref, wd_ref, bd_ref, o_ref,
               h_ref, c_ref, *, tt, batch):
    ti = pl.program_id(0)

    @pl.when(ti == 0)
    def _():
        h_ref[...] = jnp.zeros((batch, _H), jnp.float32)
        c_ref[...] = jnp.zeros((batch, _H), jnp.float32)

    h = h_ref[...]
    c = c_ref[...]
    for k in range(tt):
        xt = x_ref[k].astype(jnp.bfloat16)
        z = jnp.dot(xt, w_ref[...], preferred_element_type=jnp.float32)
        z = z + jnp.dot(h.astype(jnp.bfloat16), u_ref[...],
                        preferred_element_type=jnp.float32)
        z = z + b_ref[...]
        i = jax.nn.sigmoid(z[:, :_H])
        f = jax.nn.sigmoid(z[:, _H:2 * _H])
        g = jnp.tanh(z[:, 2 * _H:3 * _H])
        o = jax.nn.sigmoid(z[:, 3 * _H:])
        c = f * c + i * g
        h = o * jnp.tanh(c)
        o_ref[k, 0, :] = jax.nn.sigmoid(
            jnp.sum(h * wd_ref[...], axis=1) + bd_ref[0, 0])
    h_ref[...] = h
    c_ref[...] = c


def _lstm_fused(x, w, u, b2, wdp, bdp, tt=8):
    t, batch, e = x.shape
    grid = (t // tt,)
    out = pl.pallas_call(
        functools.partial(_lstm_body, tt=tt, batch=batch),
        grid=grid,
        in_specs=[
            pl.BlockSpec((tt, batch, e), lambda ti: (ti, 0, 0)),
            pl.BlockSpec(w.shape, lambda ti: (0, 0)),
            pl.BlockSpec(u.shape, lambda ti: (0, 0)),
            pl.BlockSpec(b2.shape, lambda ti: (0, 0)),
            pl.BlockSpec(wdp.shape, lambda ti: (0, 0)),
            pl.BlockSpec(memory_space=pltpu.SMEM),
        ],
        out_specs=pl.BlockSpec((tt, 1, batch), lambda ti: (ti, 0, 0)),
        out_shape=jax.ShapeDtypeStruct((t, 1, batch), jnp.float32),
        scratch_shapes=[
            pltpu.VMEM((batch, _H), jnp.float32),
            pltpu.VMEM((batch, _H), jnp.float32),
        ],
        compiler_params=pltpu.CompilerParams(
            dimension_semantics=("arbitrary",)),
    )(x, w, u, b2, wdp, bdp)
    return out


def kernel(reviews, emb, W, U, b, gamma, beta, moving_mean, moving_var,
           Wd, bd):
    batch, t = reviews.shape
    e = emb.shape[1]
    idx2d = jnp.transpose(reviews).reshape(-1, _ROWS_PER_STREAM)
    x = _sc_gather(emb, idx2d).reshape(t, batch, e)

    inv = gamma * lax.rsqrt(moving_var + 1e-3)
    wd0 = Wd[:, 0]
    wdp = (inv * wd0)[None, :]
    bdp = (bd[0] + jnp.sum((beta - inv * moving_mean) * wd0))[None, None]
    b2 = b[None, :]

    out = _lstm_fused(x, W.astype(jnp.bfloat16), U.astype(jnp.bfloat16),
                      b2, wdp, bdp)
    return jnp.transpose(out.reshape(t, batch), (1, 0))[..., None]


# speedup vs baseline: 22.2940x; 1.0125x over previous
import functools

import jax
import jax.numpy as jnp
from jax import lax
from jax.experimental import pallas as pl
from jax.experimental.pallas import tpu as pltpu
from jax.experimental.pallas import tpu_sc as plsc

_H = 128
_ROWS_PER_STREAM = 128
_STREAMS_PER_SUPER = 10


def _sc_gather(table, idx2d):
    n_chunks = idx2d.shape[0]
    e = table.shape[1]
    info = plsc.get_sparse_core_info()
    nc, ns = info.num_cores, info.num_subcores
    nw = nc * ns
    ch_per_w = n_chunks // nw
    n_super = ch_per_w // _STREAMS_PER_SUPER
    assert n_chunks % nw == 0 and ch_per_w % _STREAMS_PER_SUPER == 0

    mesh = plsc.VectorSubcoreMesh(core_axis_name="c", subcore_axis_name="s")
    rows_per_super = _STREAMS_PER_SUPER * _ROWS_PER_STREAM

    idx4d = idx2d.reshape(nw, n_super, _STREAMS_PER_SUPER, _ROWS_PER_STREAM)

    @functools.partial(
        pl.kernel,
        mesh=mesh,
        out_type=jax.ShapeDtypeStruct((n_chunks * _ROWS_PER_STREAM, e),
                                      jnp.float32),
        scratch_types=[
            pltpu.VMEM((_STREAMS_PER_SUPER, _ROWS_PER_STREAM), jnp.int32),
            pltpu.VMEM((rows_per_super, e), jnp.float32),
            pltpu.SemaphoreType.DMA,
        ],
        compiler_params=pltpu.CompilerParams(use_tc_tiling_on_sc=False),
    )
    def k(table_hbm, idx_hbm, out_hbm, idx_v, rows_v, sem):
        wid = lax.axis_index("s") * nc + lax.axis_index("c")
        base_ch = wid * ch_per_w

        def body(s, carry):
            ch0 = base_ch + s * _STREAMS_PER_SUPER
            pltpu.sync_copy(idx_hbm.at[wid, s], idx_v)
            copies = [
                pltpu.async_copy(
                    table_hbm.at[idx_v.at[j]],
                    rows_v.at[pl.ds(j * _ROWS_PER_STREAM, _ROWS_PER_STREAM)],
                    sem,
                )
                for j in range(_STREAMS_PER_SUPER)
            ]
            for cp in copies:
                cp.wait()
            pltpu.sync_copy(
                rows_v,
                out_hbm.at[pl.ds(ch0 * _ROWS_PER_STREAM, rows_per_super)],
            )
            return carry

        lax.fori_loop(0, n_super, body, 0)

    return k(table, idx4d)


def _lstm_body(x_ref, w_ref, u_ref, b_ref, wd_ref, bd_ref, o_ref,
               h_ref, c_ref, *, tt, batch):
    ti = pl.program_id(0)

    @pl.when(ti == 0)
    def _():
        h_ref[...] = jnp.zeros((batch, _H), jnp.float32)
        c_ref[...] = jnp.zeros((batch, _H), jnp.float32)

    h = h_ref[...]
    c = c_ref[...]
    for k in range(tt):
        xt = x_ref[k].astype(jnp.bfloat16)
        z = jnp.dot(xt, w_ref[...], preferred_element_type=jnp.float32)
        z = z + jnp.dot(h.astype(jnp.bfloat16), u_ref[...],
                        preferred_element_type=jnp.float32)
        z = z + b_ref[...]
        i = jax.nn.sigmoid(z[:, :_H])
        f = jax.nn.sigmoid(z[:, _H:2 * _H])
        g = jnp.tanh(z[:, 2 * _H:3 * _H])
        o = jax.nn.sigmoid(z[:, 3 * _H:])
        c = f * c + i * g
        h = o * jnp.tanh(c)
        o_ref[k, 0, :] = jax.nn.sigmoid(
            jnp.sum(h * wd_ref[...], axis=1) + bd_ref[0, 0])
    h_ref[...] = h
    c_ref[...] = c


def _lstm_fused(x, w, u, b2, wdp, bdp, tt=20):
    t, batch, e = x.shape
    grid = (t // tt,)
    out = pl.pallas_call(
        functools.partial(_lstm_body, tt=tt, batch=batch),
        grid=grid,
        in_specs=[
            pl.BlockSpec((tt, batch, e), lambda ti: (ti, 0, 0)),
            pl.BlockSpec(w.shape, lambda ti: (0, 0)),
            pl.BlockSpec(u.shape, lambda ti: (0, 0)),
            pl.BlockSpec(b2.shape, lambda ti: (0, 0)),
            pl.BlockSpec(wdp.shape, lambda ti: (0, 0)),
            pl.BlockSpec(memory_space=pltpu.SMEM),
        ],
        out_specs=pl.BlockSpec((tt, 1, batch), lambda ti: (ti, 0, 0)),
        out_shape=jax.ShapeDtypeStruct((t, 1, batch), jnp.float32),
        scratch_shapes=[
            pltpu.VMEM((batch, _H), jnp.float32),
            pltpu.VMEM((batch, _H), jnp.float32),
        ],
        compiler_params=pltpu.CompilerParams(
            dimension_semantics=("arbitrary",)),
    )(x, w, u, b2, wdp, bdp)
    return out


def kernel(reviews, emb, W, U, b, gamma, beta, moving_mean, moving_var,
           Wd, bd):
    batch, t = reviews.shape
    e = emb.shape[1]
    idx2d = jnp.transpose(reviews).reshape(-1, _ROWS_PER_STREAM)
    x = _sc_gather(emb, idx2d).reshape(t, batch, e)

    inv = gamma * lax.rsqrt(moving_var + 1e-3)
    wd0 = Wd[:, 0]
    wdp = (inv * wd0)[None, :]
    bdp = (bd[0] + jnp.sum((beta - inv * moving_mean) * wd0))[None, None]
    b2 = b[None, :]

    out = _lstm_fused(x, W.astype(jnp.bfloat16), U.astype(jnp.bfloat16),
                      b2, wdp, bdp)
    return jnp.transpose(out.reshape(t, batch), (1, 0))[..., None]
